# baseline (speedup 1.0000x reference)
import functools

import jax
import jax.numpy as jnp
from jax import lax
from jax.experimental import pallas as pl
from jax.experimental.pallas import tpu as pltpu
from jax.experimental.pallas import tpu_sc as plsc

B = 4
D = 128
H = 224
W = 224
NC = 2
NS = 16
NW = NC * NS
D_PER_W = D // NW
LANES = 16
HV = H // LANES
WV = W // LANES


def _pos_embed_sc(row_embed, col_embed):
    mesh = plsc.VectorSubcoreMesh(core_axis_name="c", subcore_axis_name="s")

    @functools.partial(
        pl.kernel,
        out_type=jax.ShapeDtypeStruct((B, D, H, W), jnp.float32),
        mesh=mesh,
        compiler_params=pltpu.CompilerParams(needs_layout_passes=False),
        scratch_types=[
            pltpu.VMEM((H, D), jnp.float32),
            pltpu.VMEM((D_PER_W, H), jnp.float32),
            pltpu.VMEM((D_PER_W, W), jnp.float32),
            pltpu.VMEM((H, W), jnp.float32),
        ],
    )
    def k(row_hbm, col_hbm, out_hbm, stage_v, rcols_v, ccols_v, tile_v):
        wid = lax.axis_index("s") * NC + lax.axis_index("c")
        d0 = wid * D_PER_W
        iota = lax.broadcasted_iota(jnp.int32, (LANES,), 0)

        for table_hbm, cols_v, n in ((row_hbm, rcols_v, HV),
                                     (col_hbm, ccols_v, WV)):
            pltpu.sync_copy(table_hbm.at[pl.ds(0, H)], stage_v)
            for kk in range(D_PER_W):
                d_idx = jnp.full((LANES,), d0 + kk, jnp.int32)
                for i in range(n):
                    v = plsc.load_gather(stage_v, [iota + i * LANES, d_idx])
                    cols_v[kk, pl.ds(i * LANES, LANES)] = v

        for kk in range(D_PER_W):
            cvecs = [ccols_v[kk, pl.ds(i * LANES, LANES)] for i in range(WV)]

            def body(hb, carry):
                h0 = hb * LANES
                r16 = rcols_v[kk, pl.ds(h0, LANES)]
                for j in range(LANES):
                    r = r16[j]
                    for i in range(WV):
                        tile_v[h0 + j, pl.ds(i * LANES, LANES)] = cvecs[i] + r
                return carry

            lax.fori_loop(0, HV, body, 0)
            for b in range(B):
                pltpu.sync_copy(tile_v, out_hbm.at[b, d0 + kk])

    return k(row_embed, col_embed)


def kernel(x, row_embed, col_embed):
    del x
    return _pos_embed_sc(row_embed, col_embed)


# speedup vs baseline: 1.0375x; 1.0375x over previous
import functools

import jax
import jax.numpy as jnp
from jax import lax
from jax.experimental import pallas as pl
from jax.experimental.pallas import tpu as pltpu
from jax.experimental.pallas import tpu_sc as plsc

B = 4
D = 128
H = 224
W = 224
NC = 2
NS = 16
NW = NC * NS
D_PER_W = D // NW
LANES = 16
HV = H // LANES
WV = W // LANES


def _pos_embed_sc(row_embed, col_embed):
    mesh = plsc.VectorSubcoreMesh(core_axis_name="c", subcore_axis_name="s")

    @functools.partial(
        pl.kernel,
        out_type=jax.ShapeDtypeStruct((B, D, H, W), jnp.float32),
        mesh=mesh,
        compiler_params=pltpu.CompilerParams(needs_layout_passes=False),
        scratch_types=[
            pltpu.VMEM((D_PER_W, H), jnp.float32),
            pltpu.VMEM((D_PER_W, W), jnp.float32),
            pltpu.VMEM((H, W), jnp.float32),
            pltpu.VMEM((H, W), jnp.float32),
            pltpu.SemaphoreType.DMA,
        ],
    )
    def k(row_hbm, col_hbm, out_hbm, rcols_v, ccols_v, tile_a, tile_b, sem):
        wid = lax.axis_index("s") * NC + lax.axis_index("c")
        d0 = wid * D_PER_W
        iota = lax.broadcasted_iota(jnp.int32, (LANES,), 0)
        tiles = (tile_a, tile_b)

        for table_hbm, cols_v, n in ((row_hbm, rcols_v, HV),
                                     (col_hbm, ccols_v, WV)):
            pltpu.sync_copy(table_hbm.at[pl.ds(0, H)],
                            tile_a.at[:, pl.ds(0, D)])
            for kk in range(D_PER_W):
                d_idx = jnp.full((LANES,), d0 + kk, jnp.int32)
                for i in range(n):
                    v = plsc.load_gather(tile_a, [iota + i * LANES, d_idx])
                    cols_v[kk, pl.ds(i * LANES, LANES)] = v

        def compute_tile(kk, tile_v):
            cvecs = [ccols_v[kk, pl.ds(i * LANES, LANES)] for i in range(WV)]

            def body(hb, carry):
                h0 = hb * LANES
                r16 = rcols_v[kk, pl.ds(h0, LANES)]
                for j in range(LANES):
                    r = r16[j]
                    for i in range(WV):
                        tile_v[h0 + j, pl.ds(i * LANES, LANES)] = cvecs[i] + r
                return carry

            lax.fori_loop(0, HV, body, 0)

        in_flight = [None, None]
        for kk in range(D_PER_W):
            buf = tiles[kk % 2]
            if in_flight[kk % 2] is not None:
                for c in in_flight[kk % 2]:
                    c.wait()
            compute_tile(kk, buf)
            in_flight[kk % 2] = [
                pltpu.async_copy(buf, out_hbm.at[b, d0 + kk], sem)
                for b in range(B)
            ]
        for copies in in_flight:
            for c in copies:
                c.wait()

    return k(row_embed, col_embed)


def kernel(x, row_embed, col_embed):
    del x
    return _pos_embed_sc(row_embed, col_embed)
